# Initial kernel scaffold; baseline (speedup 1.0000x reference)
import jax
import jax.numpy as jnp
from jax.experimental import pallas as pl


def kernel(source, target):
    raise NotImplementedError("write your pallas kernel here")


# speedup vs baseline: 3.0413x; 3.0413x over previous
import functools

import jax
import jax.numpy as jnp
from jax import lax
from jax.experimental import pallas as pl
from jax.experimental.pallas import tpu as pltpu
from jax.experimental.pallas import tpu_sc as plsc

N = 4096
M = 4096
NC = 2
NS = 16
L = 16
NW = NC * NS
SRC_PER_W = N // NW
NV = SRC_PER_W // L
MG = M // L

_mesh = plsc.VectorSubcoreMesh(core_axis_name="c", subcore_axis_name="s")


@functools.partial(
    pl.kernel,
    mesh=_mesh,
    out_type=jax.ShapeDtypeStruct((NW, L), jnp.float32),
    scratch_types=[
        pltpu.VMEM((SRC_PER_W,), jnp.float32),
        pltpu.VMEM((SRC_PER_W,), jnp.float32),
        pltpu.VMEM((SRC_PER_W,), jnp.float32),
        pltpu.VMEM((M,), jnp.float32),
        pltpu.VMEM((M,), jnp.float32),
        pltpu.VMEM((M,), jnp.float32),
        pltpu.VMEM((M,), jnp.float32),
        pltpu.VMEM((L,), jnp.float32),
    ],
)
def _nn_partials(sx_hbm, sy_hbm, sz_hbm, tx_hbm, ty_hbm, tz_hbm, out_hbm,
                 sx_v, sy_v, sz_v, a_v, b_v, c_v, d_v, out_v):
    wid = lax.axis_index("s") * NC + lax.axis_index("c")
    base = wid * SRC_PER_W

    pltpu.sync_copy(sx_hbm.at[pl.ds(base, SRC_PER_W)], sx_v)
    pltpu.sync_copy(sy_hbm.at[pl.ds(base, SRC_PER_W)], sy_v)
    pltpu.sync_copy(sz_hbm.at[pl.ds(base, SRC_PER_W)], sz_v)
    pltpu.sync_copy(tx_hbm, a_v)
    pltpu.sync_copy(ty_hbm, b_v)
    pltpu.sync_copy(tz_hbm, c_v)

    def prep(i, carry):
        tx = a_v[pl.ds(i * L, L)]
        ty = b_v[pl.ds(i * L, L)]
        tz = c_v[pl.ds(i * L, L)]
        d_v[pl.ds(i * L, L)] = tx * tx + ty * ty + tz * tz
        a_v[pl.ds(i * L, L)] = tx * jnp.float32(-2.0)
        b_v[pl.ds(i * L, L)] = ty * jnp.float32(-2.0)
        c_v[pl.ds(i * L, L)] = tz * jnp.float32(-2.0)
        return carry

    lax.fori_loop(0, MG, prep, jnp.int32(0), unroll=False)

    big = jnp.full((L,), 1e30, dtype=jnp.float32)
    half = NV // 2
    tot = jnp.zeros((L,), jnp.float32)

    for p in range(2):
        sxs = [sx_v[pl.ds((p * half + k) * L, L)] for k in range(half)]
        sys_ = [sy_v[pl.ds((p * half + k) * L, L)] for k in range(half)]
        szs = [sz_v[pl.ds((p * half + k) * L, L)] for k in range(half)]

        @plsc.parallel_loop(0, M, step=L, carry=tuple([big] * half))
        def body(j, accs):
            av = a_v[pl.ds(j, L)]
            bv = b_v[pl.ds(j, L)]
            cv = c_v[pl.ds(j, L)]
            dv = d_v[pl.ds(j, L)]
            accs = list(accs)
            for l in range(L):
                a = av[l]
                b = bv[l]
                c = cv[l]
                d = dv[l]
                for k in range(half):
                    t = a * sxs[k] + (b * sys_[k] + (c * szs[k] + d))
                    accs[k] = jnp.minimum(accs[k], t)
            return tuple(accs)

        accs = body

        for k in range(half):
            s2 = sxs[k] * sxs[k] + sys_[k] * sys_[k] + szs[k] * szs[k]
            tot = tot + (accs[k] + s2)
    out_v[...] = tot
    pltpu.sync_copy(out_v, out_hbm.at[wid])


def kernel(source, target):
    st = source.T
    tt = target.T
    partials = _nn_partials(st[0], st[1], st[2], tt[0], tt[1], tt[2])
    rmse = jnp.sqrt(jnp.sum(partials) / jnp.float32(N))
    return jnp.where(rmse < jnp.float32(0.1), jnp.float32(1.0),
                     jnp.float32(0.0))


# speedup vs baseline: 9.3843x; 3.0857x over previous
import functools

import jax
import jax.numpy as jnp
from jax import lax
from jax.experimental import pallas as pl
from jax.experimental.pallas import tpu as pltpu
from jax.experimental.pallas import tpu_sc as plsc

N = 4096
M = 4096
NC = 2
NS = 16
L = 16
NW = NC * NS
SRC_PER_W = N // NW
NV = SRC_PER_W // L
MG = M // L
CH = 32

_mesh = plsc.VectorSubcoreMesh(core_axis_name="c", subcore_axis_name="s")


@functools.partial(
    pl.kernel,
    mesh=_mesh,
    out_type=jax.ShapeDtypeStruct((NW, L), jnp.float32),
    scratch_types=[
        pltpu.VMEM((SRC_PER_W,), jnp.float32),
        pltpu.VMEM((SRC_PER_W,), jnp.float32),
        pltpu.VMEM((SRC_PER_W,), jnp.float32),
        pltpu.VMEM((M,), jnp.float32),
        pltpu.VMEM((M,), jnp.float32),
        pltpu.VMEM((M,), jnp.float32),
        pltpu.VMEM((M,), jnp.float32),
        pltpu.VMEM((CH * L,), jnp.float32),
        pltpu.VMEM((CH * L,), jnp.float32),
        pltpu.VMEM((CH * L,), jnp.float32),
        pltpu.VMEM((CH * L,), jnp.float32),
        pltpu.VMEM((L,), jnp.float32),
    ],
)
def _nn_partials(sx_hbm, sy_hbm, sz_hbm, tx_hbm, ty_hbm, tz_hbm, out_hbm,
                 sx_v, sy_v, sz_v, a_v, b_v, c_v, d_v,
                 sa_v, sb_v, sc_v, sd_v, out_v):
    wid = lax.axis_index("s") * NC + lax.axis_index("c")
    base = wid * SRC_PER_W

    pltpu.sync_copy(sx_hbm.at[pl.ds(base, SRC_PER_W)], sx_v)
    pltpu.sync_copy(sy_hbm.at[pl.ds(base, SRC_PER_W)], sy_v)
    pltpu.sync_copy(sz_hbm.at[pl.ds(base, SRC_PER_W)], sz_v)
    pltpu.sync_copy(tx_hbm, a_v)
    pltpu.sync_copy(ty_hbm, b_v)
    pltpu.sync_copy(tz_hbm, c_v)

    def prep(i, carry):
        tx = a_v[pl.ds(i * L, L)]
        ty = b_v[pl.ds(i * L, L)]
        tz = c_v[pl.ds(i * L, L)]
        d_v[pl.ds(i * L, L)] = tx * tx + ty * ty + tz * tz
        a_v[pl.ds(i * L, L)] = tx * jnp.float32(-2.0)
        b_v[pl.ds(i * L, L)] = ty * jnp.float32(-2.0)
        c_v[pl.ds(i * L, L)] = tz * jnp.float32(-2.0)
        return carry

    lax.fori_loop(0, MG, prep, jnp.int32(0), unroll=False)

    big = jnp.full((L,), 1e30, dtype=jnp.float32)
    sxs = [sx_v[pl.ds(k * L, L)] for k in range(NV)]
    sys_ = [sy_v[pl.ds(k * L, L)] for k in range(NV)]
    szs = [sz_v[pl.ds(k * L, L)] for k in range(NV)]

    def chunk(c, accs):
        accs = list(accs)
        base = c * CH
        for g in range(CH // L):
            av = a_v[pl.ds(base + g * L, L)]
            bv = b_v[pl.ds(base + g * L, L)]
            cv = c_v[pl.ds(base + g * L, L)]
            dv = d_v[pl.ds(base + g * L, L)]
            for l in range(L):
                t = g * L + l
                sa_v[pl.ds(t * L, L)] = jnp.broadcast_to(av[l], (L,))
                sb_v[pl.ds(t * L, L)] = jnp.broadcast_to(bv[l], (L,))
                sc_v[pl.ds(t * L, L)] = jnp.broadcast_to(cv[l], (L,))
                sd_v[pl.ds(t * L, L)] = jnp.broadcast_to(dv[l], (L,))
        for t in range(CH):
            sa = sa_v[pl.ds(t * L, L)]
            sb = sb_v[pl.ds(t * L, L)]
            sc = sc_v[pl.ds(t * L, L)]
            sd = sd_v[pl.ds(t * L, L)]
            for k in range(NV):
                tt = sa * sxs[k] + (sb * sys_[k] + (sc * szs[k] + sd))
                accs[k] = jnp.minimum(accs[k], tt)
        return tuple(accs)

    accs = lax.fori_loop(0, M // CH, chunk, tuple([big] * NV), unroll=False)

    tot = jnp.zeros((L,), jnp.float32)
    for k in range(NV):
        s2 = sxs[k] * sxs[k] + sys_[k] * sys_[k] + szs[k] * szs[k]
        tot = tot + (accs[k] + s2)
    out_v[...] = tot
    pltpu.sync_copy(out_v, out_hbm.at[wid])


def kernel(source, target):
    st = source.T
    tt = target.T
    partials = _nn_partials(st[0], st[1], st[2], tt[0], tt[1], tt[2])
    rmse = jnp.sqrt(jnp.sum(partials) / jnp.float32(N))
    return jnp.where(rmse < jnp.float32(0.1), jnp.float32(1.0),
                     jnp.float32(0.0))
